# Initial kernel scaffold; baseline (speedup 1.0000x reference)
import jax
import jax.numpy as jnp
from jax.experimental import pallas as pl


def kernel(x, emb_weight):
    raise NotImplementedError("write your pallas kernel here")


# speedup vs baseline: 2.4137x; 2.4137x over previous
import functools
import math

import jax
import jax.numpy as jnp
from jax import lax
from jax.experimental import pallas as pl
from jax.experimental.pallas import tpu as pltpu
from jax.experimental.pallas import tpu_sc as plsc

_D = 128
_SCALE = math.sqrt(float(_D))
_NC = 2
_NS = 16
_NW = _NC * _NS
_CHUNK = 128


def _make_lookup(total_rows: int):
    assert total_rows % (_NW * _CHUNK) == 0
    rows_per_w = total_rows // _NW
    n_chunks = rows_per_w // _CHUNK
    mesh = plsc.VectorSubcoreMesh(
        core_axis_name="c", subcore_axis_name="s", num_cores=_NC, num_subcores=_NS
    )

    @functools.partial(
        pl.kernel,
        mesh=mesh,
        out_type=jax.ShapeDtypeStruct((total_rows, _D), jnp.float32),
        scratch_types=[
            pltpu.VMEM((n_chunks, _CHUNK), jnp.int32),
            pltpu.VMEM((_CHUNK, _D), jnp.float32),
            pltpu.SemaphoreType.DMA,
        ],
    )
    def lookup(idx_hbm, table_hbm, out_hbm, idx_v, rows_v, sem):
        wid = lax.axis_index("s") * _NC + lax.axis_index("c")
        base = wid * rows_per_w
        pltpu.sync_copy(idx_hbm.at[wid], idx_v)

        def chunk_body(c, _):
            pltpu.async_copy(table_hbm.at[idx_v.at[c]], rows_v, sem).wait()

            def scale_row(r, _):
                for j in range(_D // 16):
                    sl = rows_v[r, pl.ds(j * 16, 16)]
                    rows_v[r, pl.ds(j * 16, 16)] = sl * _SCALE
                return _

            lax.fori_loop(0, _CHUNK, scale_row, 0, unroll=2)
            pltpu.sync_copy(rows_v, out_hbm.at[pl.ds(base + c * _CHUNK, _CHUNK)])
            return _

        lax.fori_loop(0, n_chunks, chunk_body, 0)

    return lookup


@jax.jit
def kernel(x, emb_weight):
    b, l = x.shape
    total = b * l
    idx = x.reshape(_NW, total // (_NW * _CHUNK), _CHUNK).astype(jnp.int32)
    out = _make_lookup(total)(idx, emb_weight)
    return out.reshape(b, l, _D)
